# Initial kernel scaffold; baseline (speedup 1.0000x reference)
import jax
import jax.numpy as jnp
from jax.experimental import pallas as pl


def kernel(x, W_down, b_down, W_in, b_in, codebook, W_out, b_out, W_up, b_up):
    raise NotImplementedError("write your pallas kernel here")


# speedup vs baseline: 1.6748x; 1.6748x over previous
import jax
import jax.numpy as jnp
from jax.experimental import pallas as pl
from jax.experimental.pallas import tpu as pltpu

B = 64
N = 1024
IN_FEAT = 192
HIDDEN = 128
CODE_DIM = 32
CODEBOOK_SIZE = 256
OUT_FEAT = 192

TB = 2048
BN = B * N
GRID = BN // TB


def _body(x_ref, wd_ref, bd_ref, wi_ref, bi_ref, cb_ref, wo_ref, bo_ref,
          wu_ref, bu_ref, out_ref, idx_ref, table_ref):
    def mm(a, b, dims):
        return jax.lax.dot_general(a.astype(jnp.bfloat16),
                                   b.astype(jnp.bfloat16), dims,
                                   preferred_element_type=jnp.float32)

    @pl.when(pl.program_id(0) == 0)
    def _():
        cb = cb_ref[...]
        t0 = mm(cb, wo_ref[...], (((1,), (1,)), ((), ()))) + bo_ref[...]
        t1 = mm(t0, wu_ref[...], (((1,), (1,)), ((), ()))) + bu_ref[...]
        table_ref[...] = jnp.clip(t1, -1.0, 1.0)

    x = x_ref[...]
    h = mm(x, wd_ref[...], (((1,), (1,)), ((), ()))) + bd_ref[...]
    z = mm(h, wi_ref[...], (((1,), (1,)), ((), ()))) + bi_ref[...]
    cb = cb_ref[...]
    scores = mm(z, cb, (((1,), (1,)), ((), ())))
    zz = jnp.sum(z * z, axis=1, keepdims=True)
    c2 = jax.lax.dot_general(jnp.ones((1, CODE_DIM), jnp.float32), cb * cb,
                             (((1,), (1,)), ((), ())),
                             precision=jax.lax.Precision.HIGHEST,
                             preferred_element_type=jnp.float32)
    dist = (zz - 2.0 * scores) + c2
    dmin = jnp.min(dist, axis=1, keepdims=True)
    lane = jax.lax.broadcasted_iota(jnp.int32, dist.shape, 1)
    idx = jnp.min(jnp.where(dist == dmin, lane, CODEBOOK_SIZE), axis=1,
                  keepdims=True)
    idx_ref[...] = idx
    onehot = (lane == idx).astype(jnp.float32)
    out_ref[...] = jax.lax.dot_general(onehot, table_ref[...],
                                       (((1,), (0,)), ((), ())),
                                       preferred_element_type=jnp.float32)


def kernel(x, W_down, b_down, W_in, b_in, codebook, W_out, b_out, W_up, b_up):
    xf = x.reshape(BN, IN_FEAT)
    full = lambda shape: pl.BlockSpec(shape, lambda i: (0,) * len(shape))
    out, idx = pl.pallas_call(
        _body,
        grid=(GRID,),
        in_specs=[
            pl.BlockSpec((TB, IN_FEAT), lambda i: (i, 0)),
            full((HIDDEN, IN_FEAT)),
            full((1, HIDDEN)),
            full((CODE_DIM, HIDDEN)),
            full((1, CODE_DIM)),
            full((CODEBOOK_SIZE, CODE_DIM)),
            full((HIDDEN, CODE_DIM)),
            full((1, HIDDEN)),
            full((OUT_FEAT, HIDDEN)),
            full((1, OUT_FEAT)),
        ],
        out_specs=[
            pl.BlockSpec((TB, OUT_FEAT), lambda i: (i, 0)),
            pl.BlockSpec((TB, 1), lambda i: (i, 0)),
        ],
        out_shape=[
            jax.ShapeDtypeStruct((BN, OUT_FEAT), jnp.float32),
            jax.ShapeDtypeStruct((BN, 1), jnp.int32),
        ],
        scratch_shapes=[pltpu.VMEM((CODEBOOK_SIZE, OUT_FEAT), jnp.float32)],
    )(xf, W_down, b_down.reshape(1, HIDDEN), W_in, b_in.reshape(1, CODE_DIM),
      codebook, W_out, b_out.reshape(1, HIDDEN), W_up, b_up.reshape(1, OUT_FEAT))
    out = out.reshape(B, N, OUT_FEAT)
    indices = idx.reshape(B, N)
    commit_loss = jnp.zeros((), dtype=jnp.float32)
    return (out, indices, commit_loss)
